# baseline (speedup 1.0000x reference)
import functools

import jax
import jax.numpy as jnp
from jax import lax
from jax.experimental import pallas as pl
from jax.experimental.pallas import tpu as pltpu
from jax.experimental.pallas import tpu_sc as plsc

HIDDEN = 768
WIDTH = 32
NUM_POS = WIDTH * WIDTH + 1
N = 262144

NC, NS, L = 2, 16, 16
NW = NC * NS
BPW = N // NW
NBUF = 4
LEAD = 2
CHUNK = 16
NCHUNK = BPW // CHUNK
GROUPS = HIDDEN // 32
HPACK = HIDDEN // 2

_mesh = plsc.VectorSubcoreMesh(core_axis_name="c", subcore_axis_name="s")


@functools.partial(
    pl.kernel,
    out_type=jax.ShapeDtypeStruct((N, HIDDEN), jnp.int32),
    mesh=_mesh,
    scratch_types=[
        pltpu.VMEM((BPW,), jnp.float32),
        pltpu.VMEM((BPW,), jnp.float32),
        pltpu.VMEM((BPW,), jnp.int32),
        pltpu.VMEM((NBUF, CHUNK, HPACK), jnp.int32),
        pltpu.VMEM((NBUF, CHUNK, HIDDEN), jnp.int32),
    ] + [pltpu.SemaphoreType.DMA] * (2 * NBUF),
)
def _uv_pos_gather(x_hbm, y_hbm, table_hbm, out_hbm, x_v, y_v, idx_v,
                   grows_v, wrows_v, *sems):
    gsems = sems[:NBUF]
    wsems = sems[NBUF:]
    wid = lax.axis_index("s") * NC + lax.axis_index("c")
    base = wid * BPW

    pltpu.sync_copy(x_hbm.at[pl.ds(base, BPW)], x_v)
    pltpu.sync_copy(y_hbm.at[pl.ds(base, BPW)], y_v)

    def idx_body(j, carry):
        x = x_v[pl.ds(L * j, L)]
        y = y_v[pl.ds(L * j, L)]
        idx = (x * WIDTH).astype(jnp.int32) * WIDTH + (y * WIDTH).astype(jnp.int32) + 1
        idx_v[pl.ds(L * j, L)] = idx
        return carry

    lax.fori_loop(0, BPW // L, idx_body, 0)

    def start_gather(c, b):
        pltpu.async_copy(
            table_hbm.at[idx_v.at[pl.ds(c * CHUNK, CHUNK)]],
            grows_v.at[b],
            gsems[b],
        )

    def out_copy(c, b):
        return pltpu.make_async_copy(
            wrows_v.at[b],
            out_hbm.at[pl.ds(base + c * CHUNK, CHUNK)],
            wsems[b],
        )

    def widen_chunk(b):
        def row_body(r, carry):
            for k in range(GROUPS):
                u = grows_v[b, r, pl.ds(16 * k, 16)]
                wrows_v[b, r, pl.ds(32 * k, 16)] = u << 16
                wrows_v[b, r, pl.ds(32 * k + 16, 16)] = u & jnp.int32(-65536)
            return carry

        lax.fori_loop(0, CHUNK, row_body, 0)

    for p in range(LEAD):
        start_gather(p, p)

    def gather_body(t, carry):
        for b in range(NBUF):
            c = NBUF * t + b
            bd = (b + LEAD) % NBUF
            pltpu.make_async_copy(
                table_hbm.at[idx_v.at[pl.ds(c * CHUNK, CHUNK)]],
                grows_v.at[b],
                gsems[b],
            ).wait()

            @pl.when(c >= LEAD)
            def _():
                out_copy(c - LEAD, bd).wait()

            widen_chunk(b)
            out_copy(c, b).start()

            @pl.when(c + LEAD < NCHUNK)
            def _():
                start_gather(c + LEAD, bd)

        return carry

    lax.fori_loop(0, NCHUNK // NBUF, gather_body, 0)

    for p in range(LEAD):
        c = NCHUNK - LEAD + p
        out_copy(c, c % NBUF).wait()


def kernel(pos, positional_embeddings):
    table = positional_embeddings.reshape(NUM_POS, HIDDEN)
    t_sw = table.astype(jnp.bfloat16).reshape(NUM_POS, GROUPS, 2, 16)
    t_sw = jnp.swapaxes(t_sw, 2, 3)
    t_packed = jax.lax.bitcast_convert_type(t_sw, jnp.int32).reshape(NUM_POS, HPACK)
    out = _uv_pos_gather(pos[:, 0], pos[:, 1], t_packed)
    return jax.lax.bitcast_convert_type(out, jnp.float32)[None]


# speedup vs baseline: 1.2670x; 1.2670x over previous
import functools

import jax
import jax.numpy as jnp
from jax import lax
from jax.experimental import pallas as pl
from jax.experimental.pallas import tpu as pltpu
from jax.experimental.pallas import tpu_sc as plsc

HIDDEN = 768
WIDTH = 32
NUM_POS = WIDTH * WIDTH + 1
N = 262144

NC, NS, L = 2, 16, 16
NW = NC * NS
BPW = N // NW
NBUF = 4
LEAD = 2
CHUNK = 16
NCHUNK = BPW // CHUNK
GROUPS = HIDDEN // 32
HPACK = HIDDEN // 2

_mesh = plsc.VectorSubcoreMesh(core_axis_name="c", subcore_axis_name="s")


@functools.partial(
    pl.kernel,
    out_type=jax.ShapeDtypeStruct((N, HIDDEN), jnp.int32),
    mesh=_mesh,
    scratch_types=[
        pltpu.VMEM((BPW,), jnp.float32),
        pltpu.VMEM((BPW,), jnp.float32),
        pltpu.VMEM((BPW,), jnp.int32),
        pltpu.VMEM((NBUF, CHUNK, HPACK), jnp.int32),
        pltpu.VMEM((NBUF, CHUNK, HIDDEN), jnp.int32),
    ] + [pltpu.SemaphoreType.DMA] * (2 * NBUF),
)
def _uv_pos_gather(x_hbm, y_hbm, table_hbm, out_hbm, x_v, y_v, idx_v,
                   grows_v, wrows_v, *sems):
    gsems = sems[:NBUF]
    wsems = sems[NBUF:]
    wid = lax.axis_index("s") * NC + lax.axis_index("c")
    base = wid * BPW

    pltpu.sync_copy(x_hbm.at[pl.ds(base, BPW)], x_v)
    pltpu.sync_copy(y_hbm.at[pl.ds(base, BPW)], y_v)

    def idx_body(j, carry):
        x = x_v[pl.ds(L * j, L)]
        y = y_v[pl.ds(L * j, L)]
        idx = (x * WIDTH).astype(jnp.int32) * WIDTH + (y * WIDTH).astype(jnp.int32) + 1
        idx_v[pl.ds(L * j, L)] = idx
        return carry

    lax.fori_loop(0, BPW // L, idx_body, 0)

    def start_gather(c, b):
        pltpu.async_copy(
            table_hbm.at[idx_v.at[pl.ds(c * CHUNK, CHUNK)]],
            grows_v.at[b],
            gsems[b],
        )

    def out_copy(c, b):
        return pltpu.make_async_copy(
            wrows_v.at[b],
            out_hbm.at[pl.ds(base + c * CHUNK, CHUNK)],
            wsems[b],
        )

    def widen_chunk(b):
        @plsc.parallel_loop(0, CHUNK, step=1, unroll=2)
        def _(r):
            for k in range(GROUPS):
                u = grows_v[b, r, pl.ds(16 * k, 16)]
                wrows_v[b, r, pl.ds(32 * k, 16)] = u << 16
                wrows_v[b, r, pl.ds(32 * k + 16, 16)] = u & jnp.int32(-65536)

    for p in range(LEAD):
        start_gather(p, p)

    def gather_body(t, carry):
        for b in range(NBUF):
            c = NBUF * t + b
            bd = (b + LEAD) % NBUF
            pltpu.make_async_copy(
                table_hbm.at[idx_v.at[pl.ds(c * CHUNK, CHUNK)]],
                grows_v.at[b],
                gsems[b],
            ).wait()

            @pl.when(c >= LEAD)
            def _():
                out_copy(c - LEAD, bd).wait()

            widen_chunk(b)
            out_copy(c, b).start()

            @pl.when(c + LEAD < NCHUNK)
            def _():
                start_gather(c + LEAD, bd)

        return carry

    lax.fori_loop(0, NCHUNK // NBUF, gather_body, 0)

    for p in range(LEAD):
        c = NCHUNK - LEAD + p
        out_copy(c, c % NBUF).wait()


def kernel(pos, positional_embeddings):
    table = positional_embeddings.reshape(NUM_POS, HIDDEN)
    t_sw = table.astype(jnp.bfloat16).reshape(NUM_POS, GROUPS, 2, 16)
    t_sw = jnp.swapaxes(t_sw, 2, 3)
    t_packed = jax.lax.bitcast_convert_type(t_sw, jnp.int32).reshape(NUM_POS, HPACK)
    out = _uv_pos_gather(pos[:, 0], pos[:, 1], t_packed)
    return jax.lax.bitcast_convert_type(out, jnp.float32)[None]


# speedup vs baseline: 2.3479x; 1.8531x over previous
import functools

import jax
import jax.numpy as jnp
from jax import lax
from jax.experimental import pallas as pl
from jax.experimental.pallas import tpu as pltpu
from jax.experimental.pallas import tpu_sc as plsc

HIDDEN = 768
WIDTH = 32
NUM_POS = WIDTH * WIDTH + 1
N = 262144

NC, NS, L = 2, 16, 16
NW = NC * NS
BPW = N // NW
NBUF = 4
LEAD = 2
CHUNK = 32
NCHUNK = BPW // CHUNK

_mesh = plsc.VectorSubcoreMesh(core_axis_name="c", subcore_axis_name="s")


@functools.partial(
    pl.kernel,
    out_type=jax.ShapeDtypeStruct((N, HIDDEN), jnp.float32),
    mesh=_mesh,
    scratch_types=[
        pltpu.VMEM((BPW,), jnp.float32),
        pltpu.VMEM((BPW,), jnp.float32),
        pltpu.VMEM((BPW,), jnp.int32),
        pltpu.VMEM((NBUF, CHUNK, HIDDEN), jnp.float32),
    ] + [pltpu.SemaphoreType.DMA] * (2 * NBUF),
)
def _uv_pos_gather(x_hbm, y_hbm, table_hbm, out_hbm, x_v, y_v, idx_v, rows_v,
                   *sems):
    gsems = sems[:NBUF]
    wsems = sems[NBUF:]
    wid = lax.axis_index("s") * NC + lax.axis_index("c")
    base = wid * BPW

    pltpu.sync_copy(x_hbm.at[pl.ds(base, BPW)], x_v)
    pltpu.sync_copy(y_hbm.at[pl.ds(base, BPW)], y_v)

    def idx_body(j, carry):
        x = x_v[pl.ds(L * j, L)]
        y = y_v[pl.ds(L * j, L)]
        idx = (x * WIDTH).astype(jnp.int32) * WIDTH + (y * WIDTH).astype(jnp.int32) + 1
        idx_v[pl.ds(L * j, L)] = idx
        return carry

    lax.fori_loop(0, BPW // L, idx_body, 0)

    def start_gather(c, b):
        pltpu.async_copy(
            table_hbm.at[idx_v.at[pl.ds(c * CHUNK, CHUNK)]],
            rows_v.at[b],
            gsems[b],
        )

    def out_copy(c, b):
        return pltpu.make_async_copy(
            rows_v.at[b],
            out_hbm.at[pl.ds(base + c * CHUNK, CHUNK)],
            wsems[b],
        )

    for p in range(LEAD):
        start_gather(p, p)

    def gather_body(t, carry):
        for b in range(NBUF):
            c = NBUF * t + b
            bd = (b + LEAD) % NBUF
            pltpu.make_async_copy(
                table_hbm.at[idx_v.at[pl.ds(c * CHUNK, CHUNK)]],
                rows_v.at[b],
                gsems[b],
            ).wait()

            @pl.when(c >= LEAD)
            def _():
                out_copy(c - LEAD, bd).wait()

            @pl.when(c + LEAD < NCHUNK)
            def _():
                start_gather(c + LEAD, bd)

            out_copy(c, b).start()

        return carry

    lax.fori_loop(0, NCHUNK // NBUF, gather_body, 0)

    for p in range(LEAD):
        c = NCHUNK - LEAD + p
        out_copy(c, c % NBUF).wait()


def kernel(pos, positional_embeddings):
    table = positional_embeddings.reshape(NUM_POS, HIDDEN)
    out = _uv_pos_gather(pos[:, 0], pos[:, 1], table)
    return out[None]


# speedup vs baseline: 2.4264x; 1.0334x over previous
import functools

import jax
import jax.numpy as jnp
from jax import lax
from jax.experimental import pallas as pl
from jax.experimental.pallas import tpu as pltpu
from jax.experimental.pallas import tpu_sc as plsc

HIDDEN = 768
WIDTH = 32
NUM_POS = WIDTH * WIDTH + 1
N = 262144

NC, NS, L = 2, 16, 16
NW = NC * NS
BPW = N // NW
NBUF = 4
LEAD = 2
CHUNK = 32
NCHUNK = BPW // CHUNK

_mesh = plsc.VectorSubcoreMesh(core_axis_name="c", subcore_axis_name="s")


@functools.partial(
    pl.kernel,
    out_type=jax.ShapeDtypeStruct((N, HIDDEN), jnp.float32),
    mesh=_mesh,
    scratch_types=[
        pltpu.VMEM((BPW,), jnp.float32),
        pltpu.VMEM((BPW,), jnp.float32),
        pltpu.VMEM((BPW,), jnp.int32),
        pltpu.VMEM((NBUF, CHUNK, HIDDEN), jnp.float32),
    ] + [pltpu.SemaphoreType.DMA] * (2 * NBUF),
)
def _uv_pos_gather(x_hbm, y_hbm, table_hbm, out_hbm, x_v, y_v, idx_v, rows_v,
                   *sems):
    gsems = sems[:NBUF]
    wsems = sems[NBUF:]
    wid = lax.axis_index("s") * NC + lax.axis_index("c")
    base = wid * BPW

    pltpu.sync_copy(x_hbm.at[pl.ds(base, BPW)], x_v)
    pltpu.sync_copy(y_hbm.at[pl.ds(base, BPW)], y_v)

    def idx_body(j, carry):
        x = x_v[pl.ds(L * j, L)]
        y = y_v[pl.ds(L * j, L)]
        idx = (x * WIDTH).astype(jnp.int32) * WIDTH + (y * WIDTH).astype(jnp.int32) + 1
        idx_v[pl.ds(L * j, L)] = idx + NUM_POS * (j & 3)
        return carry

    lax.fori_loop(0, BPW // L, idx_body, 0)

    def start_gather(c, b):
        pltpu.async_copy(
            table_hbm.at[idx_v.at[pl.ds(c * CHUNK, CHUNK)]],
            rows_v.at[b],
            gsems[b],
        )

    def out_copy(c, b):
        return pltpu.make_async_copy(
            rows_v.at[b],
            out_hbm.at[pl.ds(base + c * CHUNK, CHUNK)],
            wsems[b],
        )

    for p in range(LEAD):
        start_gather(p, p)

    def gather_body(t, carry):
        for b in range(NBUF):
            c = NBUF * t + b
            bd = (b + LEAD) % NBUF
            pltpu.make_async_copy(
                table_hbm.at[idx_v.at[pl.ds(c * CHUNK, CHUNK)]],
                rows_v.at[b],
                gsems[b],
            ).wait()

            @pl.when(c >= LEAD)
            def _():
                out_copy(c - LEAD, bd).wait()

            @pl.when(c + LEAD < NCHUNK)
            def _():
                start_gather(c + LEAD, bd)

            out_copy(c, b).start()

        return carry

    lax.fori_loop(0, NCHUNK // NBUF, gather_body, 0)

    for p in range(LEAD):
        c = NCHUNK - LEAD + p
        out_copy(c, c % NBUF).wait()


def kernel(pos, positional_embeddings):
    table = positional_embeddings.reshape(NUM_POS, HIDDEN)
    table4 = jnp.tile(table, (4, 1))
    out = _uv_pos_gather(pos[:, 0], pos[:, 1], table4)
    return out[None]
